# Initial kernel scaffold; baseline (speedup 1.0000x reference)
import jax
import jax.numpy as jnp
from jax.experimental import pallas as pl


def kernel(x, u, centers, temperature):
    raise NotImplementedError("write your pallas kernel here")


# speedup vs baseline: 2.0851x; 2.0851x over previous
import functools

import jax
import jax.numpy as jnp
from jax import lax
from jax.experimental import pallas as pl
from jax.experimental.pallas import tpu as pltpu
from jax.experimental.pallas import tpu_sc as plsc

NC, NS, L = 2, 16, 16
NW = NC * NS
N = 64 * 576 * 768
PER_W = N // NW
CHUNK = 32768
N_CHUNKS = PER_W // CHUNK


@functools.partial(
    pl.kernel,
    out_type=jax.ShapeDtypeStruct((N,), jnp.float32),
    mesh=plsc.VectorSubcoreMesh(core_axis_name="c", subcore_axis_name="s"),
    scratch_types=[
        pltpu.VMEM((CHUNK,), jnp.float32),
        pltpu.VMEM((8, L), jnp.float32),
    ],
)
def _sc_quantize(x_hbm, params_hbm, out_hbm, buf, pbuf):
    wid = lax.axis_index("s") * NC + lax.axis_index("c")
    base = wid * PER_W
    pltpu.sync_copy(params_hbm, pbuf)
    c0 = pbuf[0]
    c1 = pbuf[1]
    c2 = pbuf[2]
    t01 = pbuf[3]
    t12 = pbuf[4]

    def chunk_step(g, carry):
        off = base + g * CHUNK
        pltpu.sync_copy(x_hbm.at[pl.ds(off, CHUNK)], buf)

        def step(i, c):
            xv = buf[pl.ds(i * L, L)]
            buf[pl.ds(i * L, L)] = jnp.where(
                xv <= t01, c0, jnp.where(xv <= t12, c1, c2))
            return c

        lax.fori_loop(0, CHUNK // L, step, 0, unroll=4)
        pltpu.sync_copy(buf, out_hbm.at[pl.ds(off, CHUNK)])
        return carry

    lax.fori_loop(0, N_CHUNKS, chunk_step, 0)


def kernel(x, u, centers, temperature):
    del u, temperature
    c0 = centers[0]
    c1 = centers[1]
    c2 = centers[2]
    rows = jnp.stack([c0, c1, c2, (c0 + c1) * 0.5, (c1 + c2) * 0.5,
                      jnp.float32(0), jnp.float32(0), jnp.float32(0)])
    params = jnp.broadcast_to(rows[:, None], (8, L)).astype(jnp.float32)
    out = _sc_quantize(x.reshape(-1), params)
    return out.reshape(x.shape)


# speedup vs baseline: 2.5660x; 1.2306x over previous
import functools

import jax
import jax.numpy as jnp
from jax import lax
from jax.experimental import pallas as pl
from jax.experimental.pallas import tpu as pltpu
from jax.experimental.pallas import tpu_sc as plsc

NC, NS, L = 2, 16, 16
NW = NC * NS
N = 64 * 576 * 768
PER_W = N // NW
CHUNK = 32768
N_CHUNKS = PER_W // CHUNK
NBUF = 3


@functools.partial(
    pl.kernel,
    out_type=jax.ShapeDtypeStruct((N,), jnp.float32),
    mesh=plsc.VectorSubcoreMesh(core_axis_name="c", subcore_axis_name="s"),
    scratch_types=[
        pltpu.VMEM((CHUNK,), jnp.float32),
        pltpu.VMEM((CHUNK,), jnp.float32),
        pltpu.VMEM((CHUNK,), jnp.float32),
        pltpu.VMEM((8, L), jnp.float32),
        pltpu.SemaphoreType.DMA,
        pltpu.SemaphoreType.DMA,
        pltpu.SemaphoreType.DMA,
        pltpu.SemaphoreType.DMA,
        pltpu.SemaphoreType.DMA,
        pltpu.SemaphoreType.DMA,
    ],
)
def _sc_quantize(x_hbm, params_hbm, out_hbm, b0, b1, b2, pbuf,
                 si0, si1, si2, so0, so1, so2):
    bufs = (b0, b1, b2)
    sin = (si0, si1, si2)
    sout = (so0, so1, so2)
    wid = lax.axis_index("s") * NC + lax.axis_index("c")
    base = wid * PER_W
    pltpu.sync_copy(params_hbm, pbuf)
    c0 = pbuf[0]
    c1 = pbuf[1]
    c2 = pbuf[2]
    t01 = pbuf[3]
    t12 = pbuf[4]

    @pl.loop(0, N_CHUNKS, step=NBUF)
    def _outer(g0):
        in_h = []
        for b in range(NBUF):
            off = base + (g0 + b) * CHUNK

            @pl.when(g0 > 0)
            def _drain(b=b, off=off):
                pltpu.make_async_copy(
                    bufs[b], out_hbm.at[pl.ds(off, CHUNK)], sout[b]).wait()

            in_h.append(pltpu.async_copy(
                x_hbm.at[pl.ds(off, CHUNK)], bufs[b], sin[b]))
        for b in range(NBUF):
            off = base + (g0 + b) * CHUNK
            in_h[b].wait()
            buf = bufs[b]

            @plsc.parallel_loop(0, CHUNK, step=L, unroll=8)
            def _inner(i, buf=buf):
                xv = buf[pl.ds(i, L)]
                buf[pl.ds(i, L)] = jnp.where(
                    xv <= t01, c0, jnp.where(xv <= t12, c1, c2))

            pltpu.async_copy(buf, out_hbm.at[pl.ds(off, CHUNK)], sout[b])

    for b in range(NBUF):
        pltpu.make_async_copy(
            bufs[b], out_hbm.at[pl.ds(base, CHUNK)], sout[b]).wait()


def kernel(x, u, centers, temperature):
    del u, temperature
    c0 = centers[0]
    c1 = centers[1]
    c2 = centers[2]
    rows = jnp.stack([c0, c1, c2, (c0 + c1) * 0.5, (c1 + c2) * 0.5,
                      jnp.float32(0), jnp.float32(0), jnp.float32(0)])
    params = jnp.broadcast_to(rows[:, None], (8, L)).astype(jnp.float32)
    out = _sc_quantize(x.reshape(-1), params)
    return out.reshape(x.shape)


# speedup vs baseline: 7.0073x; 2.7308x over previous
import functools

import jax
import jax.numpy as jnp
from jax import lax
from jax.experimental import pallas as pl
from jax.experimental.pallas import tpu as pltpu
from jax.experimental.pallas import tpu_sc as plsc

NC, NS, L = 2, 16, 16
NW = NC * NS
D = 768
ROWS = 64 * 576
PER_W = ROWS // NW
CROWS = 48
N_CHUNKS = PER_W // CROWS
NBUF = 3


@functools.partial(
    pl.kernel,
    out_type=jax.ShapeDtypeStruct((ROWS, D), jnp.float32),
    mesh=plsc.VectorSubcoreMesh(core_axis_name="c", subcore_axis_name="s"),
    scratch_types=[
        pltpu.VMEM((CROWS, D), jnp.float32),
        pltpu.VMEM((CROWS, D), jnp.float32),
        pltpu.VMEM((CROWS, D), jnp.float32),
        pltpu.VMEM((8, L), jnp.float32),
        pltpu.SemaphoreType.DMA,
        pltpu.SemaphoreType.DMA,
        pltpu.SemaphoreType.DMA,
        pltpu.SemaphoreType.DMA,
        pltpu.SemaphoreType.DMA,
        pltpu.SemaphoreType.DMA,
    ],
)
def _sc_quantize(x_hbm, params_hbm, out_hbm, b0, b1, b2, pbuf,
                 si0, si1, si2, so0, so1, so2):
    bufs = (b0, b1, b2)
    sin = (si0, si1, si2)
    sout = (so0, so1, so2)
    wid = lax.axis_index("s") * NC + lax.axis_index("c")
    base = wid * PER_W
    pltpu.sync_copy(params_hbm, pbuf)
    c0 = pbuf[0]
    c1 = pbuf[1]
    c2 = pbuf[2]
    t01 = pbuf[3]
    t12 = pbuf[4]

    @pl.loop(0, N_CHUNKS, step=NBUF)
    def _outer(g0):
        in_h = []
        for b in range(NBUF):
            row0 = base + (g0 + b) * CROWS

            @pl.when(g0 > 0)
            def _drain(b=b, row0=row0):
                pltpu.make_async_copy(
                    bufs[b], out_hbm.at[pl.ds(row0, CROWS)], sout[b]).wait()

            in_h.append(pltpu.async_copy(
                x_hbm.at[pl.ds(row0, CROWS)], bufs[b], sin[b]))
        for b in range(NBUF):
            row0 = base + (g0 + b) * CROWS
            in_h[b].wait()
            buf = bufs[b]

            @pl.loop(0, CROWS)
            def _row(r, buf=buf):
                @plsc.parallel_loop(0, D, step=L, unroll=8)
                def _col(cc):
                    xv = buf[r, pl.ds(cc, L)]
                    buf[r, pl.ds(cc, L)] = jnp.where(
                        xv <= t01, c0, jnp.where(xv <= t12, c1, c2))

            pltpu.async_copy(buf, out_hbm.at[pl.ds(row0, CROWS)], sout[b])

    for b in range(NBUF):
        pltpu.make_async_copy(
            bufs[b], out_hbm.at[pl.ds(base, CROWS)], sout[b]).wait()


def kernel(x, u, centers, temperature):
    del u, temperature
    c0 = centers[0]
    c1 = centers[1]
    c2 = centers[2]
    rows = jnp.stack([c0, c1, c2, (c0 + c1) * 0.5, (c1 + c2) * 0.5,
                      jnp.float32(0), jnp.float32(0), jnp.float32(0)])
    params = jnp.broadcast_to(rows[:, None], (8, L)).astype(jnp.float32)
    out = _sc_quantize(x.reshape(ROWS, D), params)
    return out.reshape(x.shape)


# speedup vs baseline: 7.3092x; 1.0431x over previous
import functools

import jax
import jax.numpy as jnp
from jax import lax
from jax.experimental import pallas as pl
from jax.experimental.pallas import tpu as pltpu
from jax.experimental.pallas import tpu_sc as plsc

NC, NS, L = 2, 16, 16
NW = NC * NS
D = 768
ROWS = 64 * 576
PER_W = ROWS // NW
CROWS = 48
N_CHUNKS = PER_W // CROWS
NBUF = 3


@functools.partial(
    pl.kernel,
    out_type=jax.ShapeDtypeStruct((ROWS, D), jnp.float32),
    mesh=plsc.VectorSubcoreMesh(core_axis_name="c", subcore_axis_name="s"),
    scratch_types=[
        pltpu.VMEM((CROWS, D), jnp.float32),
        pltpu.VMEM((CROWS, D), jnp.float32),
        pltpu.VMEM((CROWS, D), jnp.float32),
        pltpu.VMEM((8, L), jnp.float32),
        pltpu.SemaphoreType.DMA,
        pltpu.SemaphoreType.DMA,
        pltpu.SemaphoreType.DMA,
        pltpu.SemaphoreType.DMA,
        pltpu.SemaphoreType.DMA,
        pltpu.SemaphoreType.DMA,
    ],
)
def _sc_quantize(x_hbm, params_hbm, out_hbm, b0, b1, b2, pbuf,
                 si0, si1, si2, so0, so1, so2):
    bufs = (b0, b1, b2)
    sin = (si0, si1, si2)
    sout = (so0, so1, so2)
    wid = lax.axis_index("s") * NC + lax.axis_index("c")
    base = wid * PER_W
    pltpu.sync_copy(params_hbm, pbuf)
    c0 = pbuf[0]
    c1 = pbuf[1]
    c2 = pbuf[2]
    t01 = pbuf[3]
    t12 = pbuf[4]

    @pl.loop(0, N_CHUNKS, step=NBUF)
    def _outer(g0):
        in_h = []
        for b in range(NBUF):
            row0 = base + (g0 + b) * CROWS

            @pl.when(g0 > 0)
            def _drain(b=b, row0=row0):
                pltpu.make_async_copy(
                    bufs[b], out_hbm.at[pl.ds(row0, CROWS)], sout[b]).wait()

            in_h.append(pltpu.async_copy(
                x_hbm.at[pl.ds(row0, CROWS)], bufs[b], sin[b]))
        for b in range(NBUF):
            row0 = base + (g0 + b) * CROWS
            in_h[b].wait()
            buf = bufs[b]

            @plsc.parallel_loop(0, CROWS, step=1, unroll=2)
            def _row(r, buf=buf):
                for c in range(0, D, L):
                    xv = buf[r, pl.ds(c, L)]
                    buf[r, pl.ds(c, L)] = jnp.where(
                        xv <= t01, c0, jnp.where(xv <= t12, c1, c2))

            pltpu.async_copy(buf, out_hbm.at[pl.ds(row0, CROWS)], sout[b])

    last0 = base + (N_CHUNKS - NBUF) * CROWS
    for b in range(NBUF):
        pltpu.make_async_copy(
            bufs[b], out_hbm.at[pl.ds(last0 + b * CROWS, CROWS)],
            sout[b]).wait()


def kernel(x, u, centers, temperature):
    del u, temperature
    c0 = centers[0]
    c1 = centers[1]
    c2 = centers[2]
    rows = jnp.stack([c0, c1, c2, (c0 + c1) * 0.5, (c1 + c2) * 0.5,
                      jnp.float32(0), jnp.float32(0), jnp.float32(0)])
    params = jnp.broadcast_to(rows[:, None], (8, L)).astype(jnp.float32)
    out = _sc_quantize(x.reshape(ROWS, D), params)
    return out.reshape(x.shape)
